# baseline (speedup 1.0000x reference)
import functools

import jax
import jax.numpy as jnp
from jax import lax
from jax.experimental import pallas as pl
from jax.experimental.pallas import tpu as pltpu
from jax.experimental.pallas import tpu_sc as plsc

_CH0, _CH1, _CH2 = 128, 64, 16
_KS, _DIM = 5, 3
_K = _KS ** _DIM
_N = 10000
_E = 320000
_NC, _NS = 2, 16
_NW = _NC * _NS
_EPW = _E // _NW
_CE = 2000
_NCHUNK = _EPW // _CE
_GPC = _CE // 16
_NP = 10240
_RPT = _NP // _NS


def _elu(x):
    return jnp.where(x > 0, x, jnp.exp(jnp.minimum(x, 0.0)) - 1.0)


def _encedge_body(x_ref, w1t_ref, b1_ref, w2t_ref, b2_ref, wf_ref, root_ref,
                  bsp_ref, a0_ref, a1_ref, a2_ref, src_ref,
                  z_ref, r_ref, ridx_ref, basis_ref):
    h = _elu(x_ref[...] @ w1t_ref[...] + b1_ref[...])
    h = _elu(h @ w2t_ref[...] + b2_ref[...])
    z_ref[...] = h @ wf_ref[...]
    r_ref[...] = h @ root_ref[...] + bsp_ref[...]
    fr, lo = [], []
    for aref in (a0_ref, a1_ref, a2_ref):
        v = aref[:, 0, :] * float(_KS - 1)
        l = jnp.floor(v)
        fr.append(v - l)
        lo.append(l.astype(jnp.int32))
    base = src_ref[:, 0, :] * _K
    rows_i, rows_b = [], []
    for s in range(8):
        b = jnp.ones_like(fr[0])
        wi = jnp.zeros_like(base)
        for d in range(_DIM):
            bit = (s >> d) & 1
            b = b * (fr[d] if bit else (1.0 - fr[d]))
            wi = wi + jnp.minimum(lo[d] + bit, _KS - 1) * (_KS ** d)
        rows_i.append(base + wi)
        rows_b.append(b)
    ridx_ref[...] = jnp.stack(rows_i, axis=1)
    basis_ref[...] = jnp.stack(rows_b, axis=1)


def _enc_and_edges(xf, w1t, b1, w2t, b2, wf, root, bsp, a0, a1, a2, src):
    grid = 10
    bn = _N // grid
    nch = _E // _CE
    mb = nch // grid
    r3 = lambda a: a.reshape(nch, 1, _CE)
    return pl.pallas_call(
        _encedge_body,
        grid=(grid,),
        in_specs=[
            pl.BlockSpec((bn, _CH0), lambda i: (i, 0)),
            pl.BlockSpec((_CH0, _CH1), lambda i: (0, 0)),
            pl.BlockSpec((1, _CH1), lambda i: (0, 0)),
            pl.BlockSpec((_CH1, _CH2), lambda i: (0, 0)),
            pl.BlockSpec((1, _CH2), lambda i: (0, 0)),
            pl.BlockSpec((_CH2, _K * _CH2), lambda i: (0, 0)),
            pl.BlockSpec((_CH2, _CH2), lambda i: (0, 0)),
            pl.BlockSpec((1, _CH2), lambda i: (0, 0)),
        ] + [pl.BlockSpec((mb, 1, _CE), lambda i: (i, 0, 0))] * 4,
        out_specs=[
            pl.BlockSpec((bn, _K * _CH2), lambda i: (i, 0)),
            pl.BlockSpec((bn, _CH2), lambda i: (i, 0)),
            pl.BlockSpec((mb, 8, _CE), lambda i: (i, 0, 0)),
            pl.BlockSpec((mb, 8, _CE), lambda i: (i, 0, 0)),
        ],
        out_shape=[
            jax.ShapeDtypeStruct((_N, _K * _CH2), jnp.float32),
            jax.ShapeDtypeStruct((_N, _CH2), jnp.float32),
            jax.ShapeDtypeStruct((nch, 8, _CE), jnp.int32),
            jax.ShapeDtypeStruct((nch, 8, _CE), jnp.float32),
        ],
    )(xf, w1t, b1, w2t, b2, wf, root, bsp, r3(a0), r3(a1), r3(a2), r3(src))


_mesh = plsc.VectorSubcoreMesh(
    core_axis_name="c", subcore_axis_name="s", num_cores=_NC, num_subcores=_NS
)


@functools.partial(
    pl.kernel,
    out_type=[jax.ShapeDtypeStruct((_NC, _NP, _CH2), jnp.float32),
              jax.ShapeDtypeStruct((_NC, _NP), jnp.float32)],
    mesh=_mesh,
    compiler_params=pltpu.CompilerParams(use_tc_tiling_on_sc=False),
    scratch_types=[
        pltpu.VMEM((8, _CE), jnp.int32),
        pltpu.VMEM((8, _CE), jnp.float32),
        pltpu.VMEM((_CE,), jnp.int32),
        pltpu.VMEM((5, 128), jnp.int32),
        pltpu.VMEM((5, 128), jnp.int32),
        pltpu.VMEM((640, _CH2), jnp.float32),
        pltpu.VMEM((640, _CH2), jnp.float32),
        pltpu.VMEM((80,), jnp.int32),
        pltpu.VMEM((80, _CH2), jnp.float32),
        pltpu.VMEM((80,), jnp.float32),
        pltpu.VMEM((128, _CH2), jnp.float32),
        pltpu.VMEM((640,), jnp.float32),
        pltpu.VMEM_SHARED((_NP, _CH2), jnp.float32),
        pltpu.VMEM_SHARED((_NP,), jnp.float32),
        pltpu.SemaphoreType.DMA,
        pltpu.SemaphoreType.DMA,
    ],
)
def _sc_spline(z_ref, ridx_ref, basis_ref, dst_ref, out_ref, deg_ref,
               ridx_c, basis_c, dst_c, idx_a, idx_b, rows_a, rows_b,
               dst_v, out_buf, ones_v, zbuf, zbufd, agg_sh, deg_sh,
               sem_a, sem_b):
    cid = lax.axis_index("c")
    sid = lax.axis_index("s")
    wid = cid * _NS + sid
    iota = lax.iota(jnp.int32, 16)
    zero16 = jnp.zeros((16,), jnp.float32)

    def _zb(r, carry):
        zbuf[r, pl.ds(0, 16)] = zero16
        return carry

    lax.fori_loop(0, 128, _zb, 0)

    def _zd(r, carry):
        zbufd[pl.ds(r * 16, 16)] = zero16
        return carry

    lax.fori_loop(0, 40, _zd, 0)

    def _ov(r, carry):
        ones_v[pl.ds(r * 16, 16)] = zero16 + 1.0
        return carry

    lax.fori_loop(0, 5, _ov, 0)

    def _za(i, carry):
        pltpu.sync_copy(zbuf, agg_sh.at[pl.ds(sid * _RPT + i * 128, 128)])
        return carry

    lax.fori_loop(0, _RPT // 128, _za, 0)
    pltpu.sync_copy(zbufd, deg_sh.at[pl.ds(sid * _RPT, _RPT)])

    plsc.subcore_barrier()

    def _chunk(ch, carry):
        j = wid * _NCHUNK + ch
        e0 = wid * _EPW + ch * _CE
        pltpu.sync_copy(ridx_ref.at[j], ridx_c)
        pltpu.sync_copy(basis_ref.at[j], basis_c)
        pltpu.sync_copy(dst_ref.at[pl.ds(e0, _CE)], dst_c)

        def _fire(sb, idxr, rowsr, semr):
            eb = sb * 80
            for g in range(5):
                for s in range(8):
                    idxr[g, pl.ds(s * 16, 16)] = ridx_c[s, pl.ds(eb + g * 16, 16)]
            for g in range(5):
                pltpu.async_copy(z_ref.at[idxr.at[g]],
                                 rowsr.at[pl.ds(g * 128, 128)], semr)

        def _drain(idxr, rowsr, semr):
            for g in range(5):
                pltpu.make_async_copy(z_ref.at[idxr.at[g]],
                                      rowsr.at[pl.ds(g * 128, 128)], semr).wait()

        def _proc(sb, rowsr):
            eb = sb * 80
            for g in range(5):
                gb = eb + g * 16
                bvecs = [basis_c[s, pl.ds(gb, 16)] for s in range(8)]
                for e in range(16):
                    acc = zero16
                    for s in range(8):
                        acc = acc + bvecs[s][e] * rowsr[g * 128 + s * 16 + e,
                                                        pl.ds(0, _CH2)]
                    out_buf[g * 16 + e, pl.ds(0, _CH2)] = acc
                dst_v[pl.ds(g * 16, 16)] = dst_c[pl.ds(gb, 16)]
            pltpu.sync_copy(out_buf, agg_sh.at[dst_v], add=True)
            pltpu.sync_copy(ones_v, deg_sh.at[dst_v], add=True)

        _fire(0, idx_a, rows_a, sem_a)

        def _pair(i, pcarry):
            sa = 2 * i
            _fire(sa + 1, idx_b, rows_b, sem_b)
            _drain(idx_a, rows_a, sem_a)
            _proc(sa, rows_a)
            _fire(sa + 2, idx_a, rows_a, sem_a)
            _drain(idx_b, rows_b, sem_b)
            _proc(sa + 1, rows_b)
            return pcarry

        lax.fori_loop(0, 12, _pair, 0)
        _drain(idx_a, rows_a, sem_a)
        _proc(24, rows_a)
        return carry

    lax.fori_loop(0, _NCHUNK, _chunk, 0)

    plsc.subcore_barrier()
    row0 = sid * _RPT
    pltpu.sync_copy(agg_sh.at[pl.ds(row0, _RPT)],
                    out_ref.at[cid, pl.ds(row0, _RPT)])
    pltpu.sync_copy(deg_sh.at[pl.ds(row0, _RPT)],
                    deg_ref.at[cid, pl.ds(row0, _RPT)])


def _dec_body(aggp_ref, degp_ref, r_ref, w1t_ref, b1_ref, w2t_ref, b2_ref,
              o_ref):
    ap = aggp_ref[...]
    feat = ap[0] + ap[1]
    dp = degp_ref[...]
    deg = dp[0] + dp[1]
    g = _elu(feat / jnp.maximum(deg, 1.0) + r_ref[...])
    h1 = _elu(g @ w1t_ref[...] + b1_ref[...])
    o_ref[...] = jnp.tanh(h1 @ w2t_ref[...] + b2_ref[...])


def _decode(aggp, degp, r, w1t, b1, w2t, b2):
    bn = 400
    grid = _N // bn
    return pl.pallas_call(
        _dec_body,
        grid=(grid,),
        in_specs=[
            pl.BlockSpec((_NC, bn, _CH2), lambda i: (0, i, 0)),
            pl.BlockSpec((_NC, bn, 1), lambda i: (0, i, 0)),
            pl.BlockSpec((bn, _CH2), lambda i: (i, 0)),
            pl.BlockSpec((_CH2, _CH1), lambda i: (0, 0)),
            pl.BlockSpec((1, _CH1), lambda i: (0, 0)),
            pl.BlockSpec((_CH1, _CH0), lambda i: (0, 0)),
            pl.BlockSpec((1, _CH0), lambda i: (0, 0)),
        ],
        out_specs=pl.BlockSpec((bn, _CH0), lambda i: (i, 0)),
        out_shape=jax.ShapeDtypeStruct((_N, _CH0), jnp.float32),
    )(aggp, degp, r, w1t, b1, w2t, b2)


def kernel(t, x, edge_index, edge_attr, enc_w1, enc_b1, enc_w2, enc_b2,
           sp_weight, sp_root, sp_bias, dec_w1, dec_b1, dec_w2, dec_b2):
    nb, vn, c0 = x.shape
    xf = x.reshape(nb * vn, c0)
    wf = sp_weight.transpose(1, 0, 2).reshape(_CH2, _K * _CH2)
    z, r, ridx, basis = _enc_and_edges(
        xf, enc_w1.T, enc_b1.reshape(1, -1), enc_w2.T, enc_b2.reshape(1, -1),
        wf, sp_root, sp_bias.reshape(1, -1), edge_attr[:, 0], edge_attr[:, 1],
        edge_attr[:, 2], edge_index[0])
    aggp, degp = _sc_spline(z.reshape(_N * _K, _CH2), ridx, basis,
                            edge_index[1])
    out = _decode(aggp, degp.reshape(_NC, _NP, 1), r, dec_w1.T,
                  dec_b1.reshape(1, -1), dec_w2.T, dec_b2.reshape(1, -1))
    return out.reshape(nb, vn, c0)


# speedup vs baseline: 1.0267x; 1.0267x over previous
import functools

import jax
import jax.numpy as jnp
from jax import lax
from jax.experimental import pallas as pl
from jax.experimental.pallas import tpu as pltpu
from jax.experimental.pallas import tpu_sc as plsc

_CH0, _CH1, _CH2 = 128, 64, 16
_KS, _DIM = 5, 3
_K = _KS ** _DIM
_N = 10000
_E = 320000
_NC, _NS = 2, 16
_NW = _NC * _NS
_EPW = _E // _NW
_CE = 2000
_NCHUNK = _EPW // _CE
_GPC = _CE // 16
_NP = 10240
_RPT = _NP // _NS


def _elu(x):
    return jnp.where(x > 0, x, jnp.exp(jnp.minimum(x, 0.0)) - 1.0)


def _encedge_body(x_ref, w1t_ref, b1_ref, w2t_ref, b2_ref, wf_ref, root_ref,
                  bsp_ref, a0_ref, a1_ref, a2_ref, src_ref,
                  z_ref, r_ref, ridx_ref, basis_ref):
    h = _elu(x_ref[...] @ w1t_ref[...] + b1_ref[...])
    h = _elu(h @ w2t_ref[...] + b2_ref[...])
    z_ref[...] = h @ wf_ref[...]
    r_ref[...] = h @ root_ref[...] + bsp_ref[...]
    fr, lo = [], []
    for aref in (a0_ref, a1_ref, a2_ref):
        v = aref[:, 0, :] * float(_KS - 1)
        l = jnp.floor(v)
        fr.append(v - l)
        lo.append(l.astype(jnp.int32))
    base = src_ref[:, 0, :] * _K
    rows_i, rows_b = [], []
    for s in range(8):
        b = jnp.ones_like(fr[0])
        wi = jnp.zeros_like(base)
        for d in range(_DIM):
            bit = (s >> d) & 1
            b = b * (fr[d] if bit else (1.0 - fr[d]))
            wi = wi + jnp.minimum(lo[d] + bit, _KS - 1) * (_KS ** d)
        rows_i.append(base + wi)
        rows_b.append(b)
    ridx_ref[...] = jnp.stack(rows_i, axis=1)
    basis_ref[...] = jnp.stack(rows_b, axis=1)


def _enc_and_edges(xf, w1t, b1, w2t, b2, wf, root, bsp, a0, a1, a2, src):
    grid = 10
    bn = _N // grid
    nch = _E // _CE
    mb = nch // grid
    r3 = lambda a: a.reshape(nch, 1, _CE)
    return pl.pallas_call(
        _encedge_body,
        grid=(grid,),
        in_specs=[
            pl.BlockSpec((bn, _CH0), lambda i: (i, 0)),
            pl.BlockSpec((_CH0, _CH1), lambda i: (0, 0)),
            pl.BlockSpec((1, _CH1), lambda i: (0, 0)),
            pl.BlockSpec((_CH1, _CH2), lambda i: (0, 0)),
            pl.BlockSpec((1, _CH2), lambda i: (0, 0)),
            pl.BlockSpec((_CH2, _K * _CH2), lambda i: (0, 0)),
            pl.BlockSpec((_CH2, _CH2), lambda i: (0, 0)),
            pl.BlockSpec((1, _CH2), lambda i: (0, 0)),
        ] + [pl.BlockSpec((mb, 1, _CE), lambda i: (i, 0, 0))] * 4,
        out_specs=[
            pl.BlockSpec((bn, _K * _CH2), lambda i: (i, 0)),
            pl.BlockSpec((bn, _CH2), lambda i: (i, 0)),
            pl.BlockSpec((mb, 8, _CE), lambda i: (i, 0, 0)),
            pl.BlockSpec((mb, 8, _CE), lambda i: (i, 0, 0)),
        ],
        out_shape=[
            jax.ShapeDtypeStruct((_N, _K * _CH2), jnp.float32),
            jax.ShapeDtypeStruct((_N, _CH2), jnp.float32),
            jax.ShapeDtypeStruct((nch, 8, _CE), jnp.int32),
            jax.ShapeDtypeStruct((nch, 8, _CE), jnp.float32),
        ],
    )(xf, w1t, b1, w2t, b2, wf, root, bsp, r3(a0), r3(a1), r3(a2), r3(src))


_mesh = plsc.VectorSubcoreMesh(
    core_axis_name="c", subcore_axis_name="s", num_cores=_NC, num_subcores=_NS
)


@functools.partial(
    pl.kernel,
    out_type=jax.ShapeDtypeStruct((_NC, _NP, 32), jnp.float32),
    mesh=_mesh,
    compiler_params=pltpu.CompilerParams(use_tc_tiling_on_sc=False),
    scratch_types=[
        pltpu.VMEM((8, _CE), jnp.int32),
        pltpu.VMEM((8, _CE), jnp.float32),
        pltpu.VMEM((_CE,), jnp.int32),
        pltpu.VMEM((5, 128), jnp.int32),
        pltpu.VMEM((5, 128), jnp.int32),
        pltpu.VMEM((640, _CH2), jnp.float32),
        pltpu.VMEM((640, _CH2), jnp.float32),
        pltpu.VMEM((80,), jnp.int32),
        pltpu.VMEM((80, 32), jnp.float32),
        pltpu.VMEM((128, 32), jnp.float32),
        pltpu.VMEM_SHARED((_NP, 32), jnp.float32),
        pltpu.SemaphoreType.DMA,
        pltpu.SemaphoreType.DMA,
    ],
)
def _sc_spline(z_ref, ridx_ref, basis_ref, dst_ref, out_ref,
               ridx_c, basis_c, dst_c, idx_a, idx_b, rows_a, rows_b,
               dst_v, out_buf, zbuf, agg_sh, sem_a, sem_b):
    cid = lax.axis_index("c")
    sid = lax.axis_index("s")
    wid = cid * _NS + sid
    iota = lax.iota(jnp.int32, 16)
    zero16 = jnp.zeros((16,), jnp.float32)

    def _zb(r, carry):
        zbuf[r, pl.ds(0, 16)] = zero16
        zbuf[r, pl.ds(16, 16)] = zero16
        return carry

    lax.fori_loop(0, 128, _zb, 0)

    def _za(i, carry):
        pltpu.sync_copy(zbuf, agg_sh.at[pl.ds(sid * _RPT + i * 128, 128)])
        return carry

    lax.fori_loop(0, _RPT // 128, _za, 0)

    one0 = jnp.where(iota == 0, 1.0, 0.0).astype(jnp.float32)

    def _ob(e, carry):
        out_buf[e, pl.ds(16, 16)] = one0
        return carry

    lax.fori_loop(0, 80, _ob, 0)

    plsc.subcore_barrier()

    def _chunk(ch, carry):
        j = wid * _NCHUNK + ch
        e0 = wid * _EPW + ch * _CE
        pltpu.sync_copy(ridx_ref.at[j], ridx_c)
        pltpu.sync_copy(basis_ref.at[j], basis_c)
        pltpu.sync_copy(dst_ref.at[pl.ds(e0, _CE)], dst_c)

        def _fire(sb, idxr, rowsr, semr):
            eb = sb * 80
            for g in range(5):
                for s in range(8):
                    idxr[g, pl.ds(s * 16, 16)] = ridx_c[s, pl.ds(eb + g * 16, 16)]
            for g in range(5):
                pltpu.async_copy(z_ref.at[idxr.at[g]],
                                 rowsr.at[pl.ds(g * 128, 128)], semr)

        def _drain(idxr, rowsr, semr):
            for g in range(5):
                pltpu.make_async_copy(z_ref.at[idxr.at[g]],
                                      rowsr.at[pl.ds(g * 128, 128)], semr).wait()

        def _proc(sb, rowsr):
            eb = sb * 80
            for g in range(5):
                gb = eb + g * 16
                bvecs = [basis_c[s, pl.ds(gb, 16)] for s in range(8)]
                for e in range(16):
                    acc = zero16
                    for s in range(8):
                        acc = acc + bvecs[s][e] * rowsr[g * 128 + s * 16 + e,
                                                        pl.ds(0, _CH2)]
                    out_buf[g * 16 + e, pl.ds(0, _CH2)] = acc
                dst_v[pl.ds(g * 16, 16)] = dst_c[pl.ds(gb, 16)]
            pltpu.sync_copy(out_buf, agg_sh.at[dst_v], add=True)

        _fire(0, idx_a, rows_a, sem_a)

        def _pair(i, pcarry):
            sa = 2 * i
            _fire(sa + 1, idx_b, rows_b, sem_b)
            _drain(idx_a, rows_a, sem_a)
            _proc(sa, rows_a)
            _fire(sa + 2, idx_a, rows_a, sem_a)
            _drain(idx_b, rows_b, sem_b)
            _proc(sa + 1, rows_b)
            return pcarry

        lax.fori_loop(0, 12, _pair, 0)
        _drain(idx_a, rows_a, sem_a)
        _proc(24, rows_a)
        return carry

    lax.fori_loop(0, _NCHUNK, _chunk, 0)

    plsc.subcore_barrier()
    row0 = sid * _RPT
    pltpu.sync_copy(agg_sh.at[pl.ds(row0, _RPT)],
                    out_ref.at[cid, pl.ds(row0, _RPT)])


def _dec_body(aggp_ref, r_ref, w1t_ref, b1_ref, w2t_ref, b2_ref, o_ref):
    ap = aggp_ref[...]
    ssum = ap[0] + ap[1]
    feat = ssum[:, :_CH2]
    deg = ssum[:, _CH2:_CH2 + 1]
    g = _elu(feat / jnp.maximum(deg, 1.0) + r_ref[...])
    h1 = _elu(g @ w1t_ref[...] + b1_ref[...])
    o_ref[...] = jnp.tanh(h1 @ w2t_ref[...] + b2_ref[...])


def _decode(aggp, r, w1t, b1, w2t, b2):
    bn = 400
    grid = _N // bn
    return pl.pallas_call(
        _dec_body,
        grid=(grid,),
        in_specs=[
            pl.BlockSpec((_NC, bn, 32), lambda i: (0, i, 0)),
            pl.BlockSpec((bn, _CH2), lambda i: (i, 0)),
            pl.BlockSpec((_CH2, _CH1), lambda i: (0, 0)),
            pl.BlockSpec((1, _CH1), lambda i: (0, 0)),
            pl.BlockSpec((_CH1, _CH0), lambda i: (0, 0)),
            pl.BlockSpec((1, _CH0), lambda i: (0, 0)),
        ],
        out_specs=pl.BlockSpec((bn, _CH0), lambda i: (i, 0)),
        out_shape=jax.ShapeDtypeStruct((_N, _CH0), jnp.float32),
    )(aggp, r, w1t, b1, w2t, b2)


def kernel(t, x, edge_index, edge_attr, enc_w1, enc_b1, enc_w2, enc_b2,
           sp_weight, sp_root, sp_bias, dec_w1, dec_b1, dec_w2, dec_b2):
    nb, vn, c0 = x.shape
    xf = x.reshape(nb * vn, c0)
    wf = sp_weight.transpose(1, 0, 2).reshape(_CH2, _K * _CH2)
    z, r, ridx, basis = _enc_and_edges(
        xf, enc_w1.T, enc_b1.reshape(1, -1), enc_w2.T, enc_b2.reshape(1, -1),
        wf, sp_root, sp_bias.reshape(1, -1), edge_attr[:, 0], edge_attr[:, 1],
        edge_attr[:, 2], edge_index[0])
    aggp = _sc_spline(z.reshape(_N * _K, _CH2), ridx, basis, edge_index[1])
    out = _decode(aggp, r, dec_w1.T, dec_b1.reshape(1, -1), dec_w2.T,
                  dec_b2.reshape(1, -1))
    return out.reshape(nb, vn, c0)
